# baseline (speedup 1.0000x reference)
import functools

import jax
import jax.numpy as jnp
from jax import lax
from jax.experimental import pallas as pl
from jax.experimental.pallas import tpu as pltpu
from jax.experimental.pallas import tpu_sc as plsc

N = 10000
NPAD = 10240
JUNK = NPAD - 1
E = 320000
NC, NS = 2, 16

CH128 = 80
NCH128 = E // (NC * NS * CH128)
CH16 = 128
NCH16 = 79
E16 = NC * NS * NCH16 * CH16
DCH = 128
DNCH = 157
EDEG = NS * DNCH * DCH

ROWS_PER_TILE = NPAD // NS
ROWS_PER_WORKER = NPAD // (NC * NS)
HID = 128
BLK = 2000


def _make_deg_kernel():
    mesh = plsc.VectorSubcoreMesh(core_axis_name="c", subcore_axis_name="s")

    @functools.partial(
        pl.kernel,
        out_type=jax.ShapeDtypeStruct((NPAD,), jnp.float32),
        mesh=mesh,
        scratch_types=[
            pltpu.VMEM((DNCH, DCH), jnp.int32),
            pltpu.VMEM((DCH,), jnp.float32),
            pltpu.VMEM((ROWS_PER_TILE,), jnp.float32),
            pltpu.VMEM((ROWS_PER_WORKER,), jnp.float32),
            pltpu.VMEM_SHARED((NPAD,), jnp.float32),
            pltpu.SemaphoreType.DMA,
            pltpu.SemaphoreType.DMA,
        ],
    )
    def deg_kernel(dst_hbm, dis_hbm, idx_v, ones_v, stage_v, dv, deg_sh,
                   s0, s1):
        ssem = (s0, s1)
        cid = lax.axis_index("c")
        sid = lax.axis_index("s")
        wid = cid * NS + sid

        pltpu.sync_copy(dst_hbm.at[sid], idx_v)
        for i in range(DCH // 16):
            ones_v[pl.ds(i * 16, 16)] = jnp.full((16,), 1.0, jnp.float32)

        def zbody(i, c):
            stage_v[pl.ds(i * 16, 16)] = jnp.zeros((16,), jnp.float32)
            return c

        lax.fori_loop(0, ROWS_PER_TILE // 16, zbody, 0)
        pltpu.sync_copy(
            stage_v, deg_sh.at[pl.ds(sid * ROWS_PER_TILE, ROWS_PER_TILE)])
        plsc.subcore_barrier()

        def scat(j, b):
            return pltpu.make_async_copy(ones_v, deg_sh.at[idx_v.at[j]],
                                         ssem[b])

        iters = (DNCH + 2) // 2
        jmax = 2 * iters - 1

        def body(i, c):
            for u in range(2):
                j = i * 2 + u

                @pl.when(j >= 2)
                def _():
                    scat(j - 2, u).wait()

                @pl.when(j < DNCH)
                def _():
                    scat(j, u).start(add=True)
            return c

        lax.fori_loop(0, iters, body, 0)
        for jj in range(jmax - 1, DNCH):
            scat(jj, jj % 2).wait()
        plsc.subcore_barrier()

        base = wid * ROWS_PER_WORKER
        pltpu.sync_copy(deg_sh.at[pl.ds(base, ROWS_PER_WORKER)], dv)
        for i in range(ROWS_PER_WORKER // 16):
            d = dv[pl.ds(i * 16, 16)] + 1.0
            y = 1.0 / d
            for _ in range(26):
                y = y * (1.5 - 0.5 * d * y * y)
            dv[pl.ds(i * 16, 16)] = y
        pltpu.sync_copy(dv, dis_hbm.at[pl.ds(base, ROWS_PER_WORKER)])

    return deg_kernel


def _make_agg_kernel(feat, chunk, nchunks, stage_didx, seed_y=False):
    mesh = plsc.VectorSubcoreMesh(core_axis_name="c", subcore_axis_name="s")

    scratch = [
        pltpu.VMEM((nchunks, chunk), jnp.int32),
        (pltpu.VMEM((nchunks, chunk), jnp.int32) if stage_didx
         else pltpu.VMEM((3, chunk), jnp.int32)),
        pltpu.VMEM((3, chunk, feat), jnp.float32),
        pltpu.VMEM_SHARED((NPAD, feat), jnp.float32),
    ] + [pltpu.SemaphoreType.DMA] * (6 if stage_didx else 9)

    @functools.partial(
        pl.kernel,
        out_type=jax.ShapeDtypeStruct((NC, NPAD, feat), jnp.float32),
        mesh=mesh,
        scratch_types=scratch,
        compiler_params=pltpu.CompilerParams(
            use_tc_tiling_on_sc=False) if feat < 128 else None,
    )
    def agg_kernel(y_hbm, src_hbm, dst_hbm, p_hbm, sidx, dbuf, rows, acc_sh,
                   *sems):
        gsem = sems[0:3]
        ssem = sems[3:6]
        dsem = sems[6:9] if not stage_didx else None
        cid = lax.axis_index("c")
        sid = lax.axis_index("s")
        base = sid * ROWS_PER_TILE
        nseg = ROWS_PER_TILE // chunk

        def seg(k):
            return pl.ds(base + k * chunk, chunk)

        ci = pltpu.make_async_copy(src_hbm.at[cid, sid], sidx, gsem[0])
        ci.start()
        if stage_didx:
            di = pltpu.make_async_copy(dst_hbm.at[cid, sid], dbuf, gsem[1])
            di.start()

        def zrow(i, c):
            for k in range(feat // 16):
                rows[0, i, pl.ds(k * 16, 16)] = jnp.zeros((16,), jnp.float32)
            return c

        lax.fori_loop(0, chunk, zrow, 0)

        def _use_y(k):
            return jnp.logical_and(cid == 0, base + (k + 1) * chunk <= N)

        for k in range(nseg + 1):
            if seed_y and k < nseg:
                sb = 1 + (k % 2)
                if k >= 2:
                    pltpu.make_async_copy(
                        rows.at[0], acc_sh.at[seg(k - 2)], ssem[k % 2]).wait()

                @pl.when(_use_y(k))
                def _():
                    pltpu.async_copy(
                        y_hbm.at[pl.ds(base + k * chunk, chunk)],
                        rows.at[sb], gsem[sb])
            if k >= 1:
                km = k - 1
                bm = km % 2
                if seed_y:
                    sbm = 1 + (km % 2)

                    @pl.when(_use_y(km))
                    def _():
                        pltpu.make_async_copy(
                            y_hbm.at[pl.ds(base + km * chunk, chunk)],
                            rows.at[sbm], gsem[sbm]).wait()
                        pltpu.async_copy(
                            rows.at[sbm], acc_sh.at[seg(km)], ssem[bm])

                    @pl.when(jnp.logical_not(_use_y(km)))
                    def _():
                        pltpu.async_copy(
                            rows.at[0], acc_sh.at[seg(km)], ssem[bm])
                else:
                    if km >= 2:
                        pltpu.make_async_copy(
                            rows.at[0], acc_sh.at[seg(km - 2)],
                            ssem[bm]).wait()
                    pltpu.async_copy(rows.at[0], acc_sh.at[seg(km)], ssem[bm])
        for k in range(max(nseg - 2, 0), nseg):
            pltpu.make_async_copy(
                rows.at[0], acc_sh.at[seg(k)], ssem[k % 2]).wait()
        ci.wait()
        if stage_didx:
            di.wait()
        plsc.subcore_barrier()

        def gat(j, b):
            return pltpu.make_async_copy(y_hbm.at[sidx.at[j]], rows.at[b],
                                         gsem[b])

        def dfetch(j, b):
            return pltpu.make_async_copy(dst_hbm.at[cid, sid, j],
                                         dbuf.at[b], dsem[b])

        def scat(j, b):
            idx = dbuf.at[j] if stage_didx else dbuf.at[b]
            return pltpu.make_async_copy(rows.at[b], acc_sh.at[idx], ssem[b])

        def body(i, c):
            for u in range(3):
                j = i * 3 + u
                ob = (u + 1) % 3

                @pl.when(jnp.logical_and(j >= 3, j < nchunks))
                def _():
                    scat(j - 3, u).wait()

                @pl.when(j < nchunks)
                def _():
                    if not stage_didx:
                        dfetch(j, u).start()
                    gat(j, u).start()

                @pl.when(jnp.logical_and(j >= 2, j < nchunks + 2))
                def _():
                    gat(j - 2, ob).wait()
                    if not stage_didx:
                        dfetch(j - 2, ob).wait()
                    scat(j - 2, ob).start(add=True)
            return c

        lax.fori_loop(0, (nchunks + 4) // 3, body, 0)
        scat(nchunks - 3, (nchunks - 3) % 3).wait()
        scat(nchunks - 2, (nchunks - 2) % 3).wait()
        scat(nchunks - 1, (nchunks - 1) % 3).wait()
        plsc.subcore_barrier()

        def din(k, b):
            return pltpu.make_async_copy(acc_sh.at[seg(k)], rows.at[b],
                                         gsem[b])

        def dout(k, b):
            return pltpu.make_async_copy(rows.at[b], p_hbm.at[cid, seg(k)],
                                         ssem[b])

        for k in range(nseg + 1):
            b = k % 2
            ob = (k - 1) % 2
            if k < nseg:
                if k >= 2:
                    dout(k - 2, b).wait()
                din(k, b).start()
            if k >= 1:
                din(k - 1, ob).wait()
                dout(k - 1, ob).start()
        dout(nseg - 2, (nseg - 2) % 2).wait()
        dout(nseg - 1, (nseg - 1) % 2).wait()

    return agg_kernel


_deg = _make_deg_kernel()
_agg128 = _make_agg_kernel(HID, CH128, NCH128, stage_didx=False, seed_y=True)
_agg16 = _make_agg_kernel(16, CH16, NCH16, stage_didx=True)


def _rank1(pos, wb):
    return pos[:, 0:1] * wb[0:1, :] + pos[:, 1:2] * wb[1:2, :]


def _tc0_body(x_ref, pos_ref, dis_ref, wa_ref, wb_ref, y_ref):
    xw = jnp.dot(x_ref[...], wa_ref[...], preferred_element_type=jnp.float32)
    xw = xw + _rank1(pos_ref[...], wb_ref[...])
    y_ref[...] = dis_ref[...] * xw


def _tcmid_body(p0_ref, p1_ref, pos_ref, dis_ref, b_ref, wa_ref,
                wb_ref, y_ref):
    dis = dis_ref[...]
    h = jax.nn.relu(dis * (p0_ref[0] + p1_ref[0]) + b_ref[...])
    xw = jnp.dot(h, wa_ref[...], preferred_element_type=jnp.float32)
    xw = xw + _rank1(pos_ref[...], wb_ref[...])
    y_ref[...] = dis * xw


def _tclast_body(p0_ref, p1_ref, pos_ref, dis_ref, b_ref, wa_ref,
                 wb_ref, y_ref):
    dis = dis_ref[...]
    h = jax.nn.relu(dis * (p0_ref[0] + p1_ref[0]) + b_ref[...])
    xw = jnp.dot(h, wa_ref[...], preferred_element_type=jnp.float32)
    xw = xw + _rank1(pos_ref[...], wb_ref[...])
    y = dis * xw
    y_ref[...] = jnp.concatenate(
        [y, jnp.zeros((BLK, 8), jnp.float32)], axis=1)


def _tcd_body(q0_ref, q1_ref, yp_ref, dis_ref, b_ref, out_ref):
    r = dis_ref[...] * (q0_ref[0] + q1_ref[0] + yp_ref[...])
    out_ref[...] = r[:, 0:8] + b_ref[...]


def _row_spec(cols):
    return pl.BlockSpec((BLK, cols), lambda i: (i, 0))


def _p_spec(core, cols):
    return pl.BlockSpec((1, BLK, cols), lambda i, c=core: (c, i, 0))


def _full_spec(r, c):
    return pl.BlockSpec((r, c), lambda i: (0, 0))


_GRID = N // BLK

_tc0 = pl.pallas_call(
    _tc0_body,
    grid=(_GRID,),
    in_specs=[_row_spec(HID), _row_spec(2), _row_spec(1),
              _full_spec(HID, HID), _full_spec(2, HID)],
    out_specs=_row_spec(HID),
    out_shape=jax.ShapeDtypeStruct((N, HID), jnp.float32),
)

_tcmid = pl.pallas_call(
    _tcmid_body,
    grid=(_GRID,),
    in_specs=[_p_spec(0, HID), _p_spec(1, HID), _row_spec(2),
              _row_spec(1), _full_spec(1, HID), _full_spec(HID, HID),
              _full_spec(2, HID)],
    out_specs=_row_spec(HID),
    out_shape=jax.ShapeDtypeStruct((N, HID), jnp.float32),
)

_tclast = pl.pallas_call(
    _tclast_body,
    grid=(_GRID,),
    in_specs=[_p_spec(0, HID), _p_spec(1, HID), _row_spec(2),
              _row_spec(1), _full_spec(1, HID), _full_spec(HID, 8),
              _full_spec(2, 8)],
    out_specs=_row_spec(16),
    out_shape=jax.ShapeDtypeStruct((N, 16), jnp.float32),
)

_tcd = pl.pallas_call(
    _tcd_body,
    grid=(_GRID,),
    in_specs=[_p_spec(0, 16), _p_spec(1, 16), _row_spec(16), _row_spec(1),
              _full_spec(1, 8)],
    out_specs=_row_spec(8),
    out_shape=jax.ShapeDtypeStruct((N, 8), jnp.float32),
)


def kernel(x, pos, edge_index, W0, b0, W1, b1, W2, b2):
    ei = edge_index.astype(jnp.int32)
    src_g = ei[0].reshape(NC, NS, NCH128, CH128)
    dst_g = ei[1].reshape(NC, NS, NCH128, CH128)
    pad16 = E16 - E
    src16 = jnp.concatenate(
        [ei[0], jnp.zeros((pad16,), jnp.int32)]).reshape(NC, NS, NCH16, CH16)
    dst16 = jnp.concatenate(
        [ei[1], jnp.full((pad16,), JUNK, jnp.int32)]
    ).reshape(NC, NS, NCH16, CH16)
    dst_deg = jnp.concatenate(
        [ei[1], jnp.full((EDEG - E,), JUNK, jnp.int32)]
    ).reshape(NS, DNCH, DCH)

    dis2 = _deg(dst_deg).reshape(NPAD, 1)

    y0 = _tc0(x, pos, dis2, W0[:HID], W0[HID:])
    p = _agg128(y0, src_g, dst_g)
    y1 = _tcmid(p, p, pos, dis2, b0.reshape(1, HID), W1[:HID], W1[HID:])
    q = _agg128(y1, src_g, dst_g)
    y2 = _tclast(q, q, pos, dis2, b1.reshape(1, HID), W2[:HID], W2[HID:])
    r = _agg16(y2, src16, dst16)
    return _tcd(r, r, y2, dis2, b2.reshape(1, 8))


# speedup vs baseline: 1.1828x; 1.1828x over previous
import functools

import jax
import jax.numpy as jnp
from jax import lax
from jax.experimental import pallas as pl
from jax.experimental.pallas import tpu as pltpu
from jax.experimental.pallas import tpu_sc as plsc

N = 10000
NPAD = 10240
JUNK = NPAD - 1
E = 320000
NC, NS = 2, 16

CH128 = 80
NCH128 = E // (NC * NS * CH128)
CH16 = 128
NCH16 = 79
E16 = NC * NS * NCH16 * CH16
DCH = 128
DNCH = 157
EDEG = NS * DNCH * DCH

ROWS_PER_TILE = NPAD // NS
ROWS_PER_WORKER = NPAD // (NC * NS)
HID = 128
BLK = 2000


def _make_deg_kernel():
    mesh = plsc.VectorSubcoreMesh(core_axis_name="c", subcore_axis_name="s")

    @functools.partial(
        pl.kernel,
        out_type=jax.ShapeDtypeStruct((NPAD,), jnp.float32),
        mesh=mesh,
        scratch_types=[
            pltpu.VMEM((DNCH, DCH), jnp.int32),
            pltpu.VMEM((DCH,), jnp.float32),
            pltpu.VMEM((ROWS_PER_TILE,), jnp.float32),
            pltpu.VMEM((ROWS_PER_WORKER,), jnp.float32),
            pltpu.VMEM_SHARED((NPAD,), jnp.float32),
            pltpu.SemaphoreType.DMA,
            pltpu.SemaphoreType.DMA,
        ],
    )
    def deg_kernel(dst_hbm, dis_hbm, idx_v, ones_v, stage_v, dv, deg_sh,
                   s0, s1):
        ssem = (s0, s1)
        cid = lax.axis_index("c")
        sid = lax.axis_index("s")
        wid = cid * NS + sid

        pltpu.sync_copy(dst_hbm.at[sid], idx_v)
        for i in range(DCH // 16):
            ones_v[pl.ds(i * 16, 16)] = jnp.full((16,), 1.0, jnp.float32)

        def zbody(i, c):
            stage_v[pl.ds(i * 16, 16)] = jnp.zeros((16,), jnp.float32)
            return c

        lax.fori_loop(0, ROWS_PER_TILE // 16, zbody, 0)
        pltpu.sync_copy(
            stage_v, deg_sh.at[pl.ds(sid * ROWS_PER_TILE, ROWS_PER_TILE)])
        plsc.subcore_barrier()

        def scat(j, b):
            return pltpu.make_async_copy(ones_v, deg_sh.at[idx_v.at[j]],
                                         ssem[b])

        iters = (DNCH + 2) // 2
        jmax = 2 * iters - 1

        def body(i, c):
            for u in range(2):
                j = i * 2 + u

                @pl.when(j >= 2)
                def _():
                    scat(j - 2, u).wait()

                @pl.when(j < DNCH)
                def _():
                    scat(j, u).start(add=True)
            return c

        lax.fori_loop(0, iters, body, 0)
        for jj in range(jmax - 1, DNCH):
            scat(jj, jj % 2).wait()
        plsc.subcore_barrier()

        base = wid * ROWS_PER_WORKER
        pltpu.sync_copy(deg_sh.at[pl.ds(base, ROWS_PER_WORKER)], dv)
        for i in range(ROWS_PER_WORKER // 16):
            d = dv[pl.ds(i * 16, 16)] + 1.0
            y = 1.0 / d
            for _ in range(26):
                y = y * (1.5 - 0.5 * d * y * y)
            dv[pl.ds(i * 16, 16)] = y
        pltpu.sync_copy(dv, dis_hbm.at[pl.ds(base, ROWS_PER_WORKER)])

    return deg_kernel


def _make_agg_kernel(feat, chunk, nchunks, stage_didx, seed_y=False):
    mesh = plsc.VectorSubcoreMesh(core_axis_name="c", subcore_axis_name="s")

    scratch = [
        pltpu.VMEM((nchunks, chunk), jnp.int32),
        (pltpu.VMEM((nchunks, chunk), jnp.int32) if stage_didx
         else pltpu.VMEM((3, chunk), jnp.int32)),
        pltpu.VMEM((3, chunk, feat), jnp.float32),
        pltpu.VMEM_SHARED((NPAD, feat), jnp.float32),
    ] + [pltpu.SemaphoreType.DMA] * (6 if stage_didx else 9)

    @functools.partial(
        pl.kernel,
        out_type=jax.ShapeDtypeStruct((NC, NPAD, feat), jnp.float32),
        mesh=mesh,
        scratch_types=scratch,
        compiler_params=pltpu.CompilerParams(
            use_tc_tiling_on_sc=False) if feat < 128 else None,
    )
    def agg_kernel(y_hbm, src_hbm, dst_hbm, p_hbm, sidx, dbuf, rows, acc_sh,
                   *sems):
        gsem = sems[0:3]
        ssem = sems[3:6]
        dsem = sems[6:9] if not stage_didx else None
        cid = lax.axis_index("c")
        sid = lax.axis_index("s")
        base = sid * ROWS_PER_TILE
        nseg = ROWS_PER_TILE // chunk

        def seg(k):
            return pl.ds(base + k * chunk, chunk)

        ci = pltpu.make_async_copy(src_hbm.at[cid, sid], sidx, gsem[0])
        ci.start()
        if stage_didx:
            di = pltpu.make_async_copy(dst_hbm.at[cid, sid], dbuf, gsem[1])
            di.start()

        def zrow(i, c):
            for k in range(feat // 16):
                rows[0, i, pl.ds(k * 16, 16)] = jnp.zeros((16,), jnp.float32)
            return c

        lax.fori_loop(0, chunk, zrow, 0)

        def _use_y(k):
            return jnp.logical_and(cid == 0, base + (k + 1) * chunk <= N)

        for k in range(nseg + 1):
            if seed_y and k < nseg:
                sb = 1 + (k % 2)
                if k >= 2:
                    pltpu.make_async_copy(
                        rows.at[0], acc_sh.at[seg(k - 2)], ssem[k % 2]).wait()

                @pl.when(_use_y(k))
                def _():
                    pltpu.async_copy(
                        y_hbm.at[pl.ds(base + k * chunk, chunk)],
                        rows.at[sb], gsem[sb])
            if k >= 1:
                km = k - 1
                bm = km % 2
                if seed_y:
                    sbm = 1 + (km % 2)

                    @pl.when(_use_y(km))
                    def _():
                        pltpu.make_async_copy(
                            y_hbm.at[pl.ds(base + km * chunk, chunk)],
                            rows.at[sbm], gsem[sbm]).wait()
                        pltpu.async_copy(
                            rows.at[sbm], acc_sh.at[seg(km)], ssem[bm])

                    @pl.when(jnp.logical_not(_use_y(km)))
                    def _():
                        pltpu.async_copy(
                            rows.at[0], acc_sh.at[seg(km)], ssem[bm])
                else:
                    if km >= 2:
                        pltpu.make_async_copy(
                            rows.at[0], acc_sh.at[seg(km - 2)],
                            ssem[bm]).wait()
                    pltpu.async_copy(rows.at[0], acc_sh.at[seg(km)], ssem[bm])
        for k in range(max(nseg - 2, 0), nseg):
            pltpu.make_async_copy(
                rows.at[0], acc_sh.at[seg(k)], ssem[k % 2]).wait()
        ci.wait()
        if stage_didx:
            di.wait()
        plsc.subcore_barrier()

        def gat(j, b):
            return pltpu.make_async_copy(y_hbm.at[sidx.at[j]], rows.at[b],
                                         gsem[b])

        def dfetch(j, b):
            return pltpu.make_async_copy(dst_hbm.at[cid, sid, j],
                                         dbuf.at[b], dsem[b])

        def scat(j, b):
            idx = dbuf.at[j] if stage_didx else dbuf.at[b]
            return pltpu.make_async_copy(rows.at[b], acc_sh.at[idx], ssem[b])

        def body(i, c):
            for u in range(3):
                j = i * 3 + u
                ob = (u + 1) % 3

                @pl.when(jnp.logical_and(j >= 3, j < nchunks))
                def _():
                    scat(j - 3, u).wait()

                @pl.when(j < nchunks)
                def _():
                    if not stage_didx:
                        dfetch(j, u).start()
                    gat(j, u).start()

                @pl.when(jnp.logical_and(j >= 2, j < nchunks + 2))
                def _():
                    gat(j - 2, ob).wait()
                    if not stage_didx:
                        dfetch(j - 2, ob).wait()
                    scat(j - 2, ob).start(add=True)
            return c

        lax.fori_loop(0, (nchunks + 4) // 3, body, 0)
        scat(nchunks - 3, (nchunks - 3) % 3).wait()
        scat(nchunks - 2, (nchunks - 2) % 3).wait()
        scat(nchunks - 1, (nchunks - 1) % 3).wait()
        plsc.subcore_barrier()

        def din(k, b):
            return pltpu.make_async_copy(acc_sh.at[seg(k)], rows.at[b],
                                         gsem[b])

        def dout(k, b):
            return pltpu.make_async_copy(rows.at[b], p_hbm.at[cid, seg(k)],
                                         ssem[b])

        for k in range(nseg + 1):
            b = k % 2
            ob = (k - 1) % 2
            if k < nseg:
                if k >= 2:
                    dout(k - 2, b).wait()
                din(k, b).start()
            if k >= 1:
                din(k - 1, ob).wait()
                dout(k - 1, ob).start()
        dout(nseg - 2, (nseg - 2) % 2).wait()
        dout(nseg - 1, (nseg - 1) % 2).wait()

    return agg_kernel


_deg = _make_deg_kernel()
_agg128 = _make_agg_kernel(HID, CH128, NCH128, stage_didx=False)
_agg16 = _make_agg_kernel(16, CH16, NCH16, stage_didx=True)


def _rank1(pos, wb):
    return pos[:, 0:1] * wb[0:1, :] + pos[:, 1:2] * wb[1:2, :]


def _tc0_body(x_ref, pos_ref, dis_ref, wa_ref, wb_ref, y_ref):
    xw = jnp.dot(x_ref[...], wa_ref[...], preferred_element_type=jnp.float32)
    xw = xw + _rank1(pos_ref[...], wb_ref[...])
    y_ref[...] = dis_ref[...] * xw


def _tcmid_body(p0_ref, p1_ref, yp_ref, pos_ref, dis_ref, b_ref, wa_ref,
                wb_ref, y_ref):
    dis = dis_ref[...]
    h = jax.nn.relu(dis * (p0_ref[0] + p1_ref[0] + yp_ref[...]) + b_ref[...])
    xw = jnp.dot(h, wa_ref[...], preferred_element_type=jnp.float32)
    xw = xw + _rank1(pos_ref[...], wb_ref[...])
    y_ref[...] = dis * xw


def _tclast_body(p0_ref, p1_ref, yp_ref, pos_ref, dis_ref, b_ref, wa_ref,
                 wb_ref, y_ref):
    dis = dis_ref[...]
    h = jax.nn.relu(dis * (p0_ref[0] + p1_ref[0] + yp_ref[...]) + b_ref[...])
    xw = jnp.dot(h, wa_ref[...], preferred_element_type=jnp.float32)
    xw = xw + _rank1(pos_ref[...], wb_ref[...])
    y = dis * xw
    y_ref[...] = jnp.concatenate(
        [y, jnp.zeros((BLK, 8), jnp.float32)], axis=1)


def _tcd_body(q0_ref, q1_ref, yp_ref, dis_ref, b_ref, out_ref):
    r = dis_ref[...] * (q0_ref[0] + q1_ref[0] + yp_ref[...])
    out_ref[...] = r[:, 0:8] + b_ref[...]


def _row_spec(cols):
    return pl.BlockSpec((BLK, cols), lambda i: (i, 0))


def _p_spec(core, cols):
    return pl.BlockSpec((1, BLK, cols), lambda i, c=core: (c, i, 0))


def _full_spec(r, c):
    return pl.BlockSpec((r, c), lambda i: (0, 0))


_GRID = N // BLK

_tc0 = pl.pallas_call(
    _tc0_body,
    grid=(_GRID,),
    in_specs=[_row_spec(HID), _row_spec(2), _row_spec(1),
              _full_spec(HID, HID), _full_spec(2, HID)],
    out_specs=_row_spec(HID),
    out_shape=jax.ShapeDtypeStruct((N, HID), jnp.float32),
)

_tcmid = pl.pallas_call(
    _tcmid_body,
    grid=(_GRID,),
    in_specs=[_p_spec(0, HID), _p_spec(1, HID), _row_spec(HID), _row_spec(2),
              _row_spec(1), _full_spec(1, HID), _full_spec(HID, HID),
              _full_spec(2, HID)],
    out_specs=_row_spec(HID),
    out_shape=jax.ShapeDtypeStruct((N, HID), jnp.float32),
)

_tclast = pl.pallas_call(
    _tclast_body,
    grid=(_GRID,),
    in_specs=[_p_spec(0, HID), _p_spec(1, HID), _row_spec(HID), _row_spec(2),
              _row_spec(1), _full_spec(1, HID), _full_spec(HID, 8),
              _full_spec(2, 8)],
    out_specs=_row_spec(16),
    out_shape=jax.ShapeDtypeStruct((N, 16), jnp.float32),
)

_tcd = pl.pallas_call(
    _tcd_body,
    grid=(_GRID,),
    in_specs=[_p_spec(0, 16), _p_spec(1, 16), _row_spec(16), _row_spec(1),
              _full_spec(1, 8)],
    out_specs=_row_spec(8),
    out_shape=jax.ShapeDtypeStruct((N, 8), jnp.float32),
)


def kernel(x, pos, edge_index, W0, b0, W1, b1, W2, b2):
    ei = edge_index.astype(jnp.int32)
    src_g = ei[0].reshape(NC, NS, NCH128, CH128)
    dst_g = ei[1].reshape(NC, NS, NCH128, CH128)
    pad16 = E16 - E
    src16 = jnp.concatenate(
        [ei[0], jnp.zeros((pad16,), jnp.int32)]).reshape(NC, NS, NCH16, CH16)
    dst16 = jnp.concatenate(
        [ei[1], jnp.full((pad16,), JUNK, jnp.int32)]
    ).reshape(NC, NS, NCH16, CH16)
    dst_deg = jnp.concatenate(
        [ei[1], jnp.full((EDEG - E,), JUNK, jnp.int32)]
    ).reshape(NS, DNCH, DCH)

    dis2 = _deg(dst_deg).reshape(NPAD, 1)

    y0 = _tc0(x, pos, dis2, W0[:HID], W0[HID:])
    p = _agg128(y0, src_g, dst_g)
    y1 = _tcmid(p, p, y0, pos, dis2, b0.reshape(1, HID), W1[:HID], W1[HID:])
    q = _agg128(y1, src_g, dst_g)
    y2 = _tclast(q, q, y1, pos, dis2, b1.reshape(1, HID), W2[:HID], W2[HID:])
    r = _agg16(y2, src16, dst16)
    return _tcd(r, r, y2, dis2, b2.reshape(1, 8))
